# Initial kernel scaffold; baseline (speedup 1.0000x reference)
import jax
import jax.numpy as jnp
from jax.experimental import pallas as pl


def kernel(x, edge_index, W1, b1, W2, b2):
    raise NotImplementedError("write your pallas kernel here")


# speedup vs baseline: 4.8692x; 4.8692x over previous
import functools

import jax
import jax.numpy as jnp
from jax import lax
from jax.experimental import pallas as pl
from jax.experimental.pallas import tpu as pltpu
from jax.experimental.pallas import tpu_sc as plsc

N = 10000
E = 320000
D_IN = 128
D_H = 256
N_CLS = 40
D2 = 64

NC = 2
NS = 16
NW = NC * NS
CH = 128
K = 80
E_PAD = NW * K * CH
N_PAD = 10240
STRIPE = N_PAD // NS
ZROWS = 16

_MESH = dict(core_axis_name="c", subcore_axis_name="s")


def _deg_pass():
    @functools.partial(
        pl.kernel,
        out_type=jax.ShapeDtypeStruct((2, NC, N_PAD), jnp.float32),
        mesh=plsc.VectorSubcoreMesh(**_MESH),
        scratch_types=[
            pltpu.VMEM((K, CH), jnp.int32),
            pltpu.VMEM((K, CH), jnp.int32),
            pltpu.VMEM((CH,), jnp.float32),
            pltpu.VMEM((STRIPE,), jnp.float32),
            pltpu.VMEM_SHARED((N_PAD,), jnp.float32),
            pltpu.VMEM_SHARED((N_PAD,), jnp.float32),
            pltpu.SemaphoreType.DMA,
            pltpu.SemaphoreType.DMA,
        ],
    )
    def deg_kernel(src3, dst3, out, sidx, didx, ones, zbuf, dego, degi, osem, isem):
        cid = lax.axis_index("c")
        sid = lax.axis_index("s")
        wid = cid * NS + sid
        pltpu.sync_copy(src3.at[wid], sidx)
        pltpu.sync_copy(dst3.at[wid], didx)

        @pl.loop(0, CH // 16)
        def _(i):
            ones[pl.ds(i * 16, 16)] = jnp.ones((16,), jnp.float32)

        @pl.loop(0, STRIPE // 16)
        def _(i):
            zbuf[pl.ds(i * 16, 16)] = jnp.zeros((16,), jnp.float32)

        pltpu.sync_copy(zbuf, dego.at[pl.ds(sid * STRIPE, STRIPE)])
        pltpu.sync_copy(zbuf, degi.at[pl.ds(sid * STRIPE, STRIPE)])
        plsc.subcore_barrier()

        @pl.loop(0, K)
        def _(j):
            pltpu.async_copy(ones, dego.at[sidx.at[j]], osem, add=True)
            pltpu.async_copy(ones, degi.at[didx.at[j]], isem, add=True)

        pltpu.make_async_copy(src3.at[wid], sidx, osem).wait()
        pltpu.make_async_copy(dst3.at[wid], didx, isem).wait()
        plsc.subcore_barrier()
        sl = pl.ds(sid * STRIPE, STRIPE)
        pltpu.sync_copy(dego.at[sl], out.at[0, cid, sl])
        pltpu.sync_copy(degi.at[sl], out.at[1, cid, sl])

    return deg_kernel


def _agg_pass(d, label):
    @functools.partial(
        pl.kernel,
        out_type=jax.ShapeDtypeStruct((NC, N_PAD, d), jnp.float32),
        mesh=plsc.VectorSubcoreMesh(**_MESH),
        scratch_types=[
            pltpu.VMEM((K, CH), jnp.int32),
            pltpu.VMEM((K, CH), jnp.int32),
            pltpu.VMEM((CH, d), jnp.float32),
            pltpu.VMEM((ZROWS, d), jnp.float32),
            pltpu.VMEM_SHARED((N_PAD, d), jnp.float32),
            pltpu.SemaphoreType.DMA,
            pltpu.SemaphoreType.DMA,
        ],
        name=label,
        compiler_params=pltpu.CompilerParams(use_tc_tiling_on_sc=False),
    )
    def agg_kernel(tbl, src3, dst3, out, sidx, didx, rows, zbuf, acc, gsem, ssem):
        cid = lax.axis_index("c")
        sid = lax.axis_index("s")
        wid = cid * NS + sid
        pltpu.sync_copy(src3.at[wid], sidx)
        pltpu.sync_copy(dst3.at[wid], didx)

        @pl.loop(0, ZROWS * d // 16)
        def _(i):
            zbuf[i // (d // 16), pl.ds((i % (d // 16)) * 16, 16)] = jnp.zeros(
                (16,), jnp.float32
            )

        @pl.loop(0, STRIPE // ZROWS)
        def _(t):
            pltpu.sync_copy(
                zbuf, acc.at[pl.ds(sid * STRIPE + t * ZROWS, ZROWS)]
            )

        plsc.subcore_barrier()

        @pl.loop(0, K)
        def _(j):
            pltpu.async_copy(tbl.at[sidx.at[j]], rows, gsem).wait()
            pltpu.async_copy(rows, acc.at[didx.at[j]], ssem, add=True).wait()

        plsc.subcore_barrier()
        sl = pl.ds(sid * STRIPE, STRIPE)
        pltpu.sync_copy(acc.at[sl], out.at[cid, sl])

    return agg_kernel


def _tc1(x_pad, degp4):
    R = 512
    grid = (N_PAD // R,)

    def body(dref, xref, xs_ref, ns_ref, nd_ref):
        do = dref[0, 0] + dref[0, 1]
        di = dref[1, 0] + dref[1, 1]
        ns = jnp.where(do > 0, lax.rsqrt(jnp.maximum(do, 1e-12)), 0.0)
        nd = jnp.where(di > 0, lax.rsqrt(jnp.maximum(di, 1e-12)), 0.0)
        ns_ref[...] = ns
        nd_ref[...] = nd
        xs_ref[...] = xref[...] * ns

    return pl.pallas_call(
        body,
        grid=grid,
        in_specs=[
            pl.BlockSpec((2, NC, R, 1), lambda i: (0, 0, i, 0)),
            pl.BlockSpec((R, D_IN), lambda i: (i, 0)),
        ],
        out_specs=[
            pl.BlockSpec((R, D_IN), lambda i: (i, 0)),
            pl.BlockSpec((R, 1), lambda i: (i, 0)),
            pl.BlockSpec((R, 1), lambda i: (i, 0)),
        ],
        out_shape=[
            jax.ShapeDtypeStruct((N_PAD, D_IN), jnp.float32),
            jax.ShapeDtypeStruct((N_PAD, 1), jnp.float32),
            jax.ShapeDtypeStruct((N_PAD, 1), jnp.float32),
        ],
    )(degp4, x_pad)


def _tc2(aggp, ndst, nsrc, W1, b1r, W2p):
    R = 512
    grid = (N_PAD // R,)

    def body(aref, ndref, nsref, w1ref, b1ref, w2ref, gref):
        a = (aref[0] + aref[1]) * ndref[...]
        h = jnp.dot(a, w1ref[...], preferred_element_type=jnp.float32)
        h = jnp.maximum(h + b1ref[...], 0.0)
        gref[...] = jnp.dot(
            h * nsref[...], w2ref[...], preferred_element_type=jnp.float32
        )

    return pl.pallas_call(
        body,
        grid=grid,
        in_specs=[
            pl.BlockSpec((NC, R, D_IN), lambda i: (0, i, 0)),
            pl.BlockSpec((R, 1), lambda i: (i, 0)),
            pl.BlockSpec((R, 1), lambda i: (i, 0)),
            pl.BlockSpec((D_IN, D_H), lambda i: (0, 0)),
            pl.BlockSpec((1, D_H), lambda i: (0, 0)),
            pl.BlockSpec((D_H, D2), lambda i: (0, 0)),
        ],
        out_specs=pl.BlockSpec((R, D2), lambda i: (i, 0)),
        out_shape=jax.ShapeDtypeStruct((N_PAD, D2), jnp.float32),
    )(aggp, ndst, nsrc, W1, b1r, W2p)


def _tc3(aggp2, ndst, b2r):
    R = 400
    grid = (N // R,)

    def body(aref, ndref, b2ref, oref):
        oref[...] = (aref[0] + aref[1]) * ndref[...] + b2ref[...]

    return pl.pallas_call(
        body,
        grid=grid,
        in_specs=[
            pl.BlockSpec((NC, R, D2), lambda i: (0, i, 0)),
            pl.BlockSpec((R, 1), lambda i: (i, 0)),
            pl.BlockSpec((1, D2), lambda i: (0, 0)),
        ],
        out_specs=pl.BlockSpec((R, D2), lambda i: (i, 0)),
        out_shape=jax.ShapeDtypeStruct((N, D2), jnp.float32),
    )(aggp2, ndst, b2r)


_DEG = _deg_pass()
_AGG128 = _agg_pass(D_IN, "agg128")
_AGG64 = _agg_pass(D2, "agg64")


def kernel(x, edge_index, W1, b1, W2, b2):
    src = edge_index[0]
    dst = edge_index[1]
    pad = jnp.full((E_PAD - E,), N_PAD - 1, jnp.int32)
    src3 = jnp.concatenate([src, pad]).reshape(NW, K, CH)
    dst3 = jnp.concatenate([dst, pad]).reshape(NW, K, CH)

    degp = _DEG(src3, dst3)
    x_pad = jnp.pad(x, ((0, N_PAD - N), (0, 0)))
    xs, nsrc, ndst = _tc1(x_pad, degp.reshape(2, NC, N_PAD, 1))

    agg1 = _AGG128(xs, src3, dst3)

    W2p = jnp.pad(W2, ((0, 0), (0, D2 - N_CLS)))
    g = _tc2(agg1, ndst, nsrc, W1, b1.reshape(1, D_H), W2p)

    agg2 = _AGG64(g, src3, dst3)

    b2r = jnp.pad(b2, (0, D2 - N_CLS)).reshape(1, D2)
    out64 = _tc3(agg2, ndst, b2r)
    return out64[:, :N_CLS]
